# baseline (speedup 1.0000x reference)
import jax
import jax.numpy as jnp
from jax.experimental import pallas as pl


_TM = 1536


def _dist_kernel(f_ref, c_ref, o_ref):
    f = f_ref[...]
    c = c_ref[...]
    f_sq = jnp.sum(f * f, axis=1, keepdims=True)
    c_sq = jnp.sum(c * c, axis=1, keepdims=True).reshape(1, -1)
    fneg = (-2.0 * f).astype(jnp.bfloat16)
    cross = jax.lax.dot_general(
        fneg, c.astype(jnp.bfloat16),
        dimension_numbers=(((1,), (1,)), ((), ())),
        preferred_element_type=jnp.float32)
    o_ref[...] = cross + f_sq + c_sq


def kernel(x, Ck):
    B, S, D = x.shape
    K = Ck.shape[1]
    M = B * S
    f = x.reshape(M, D)
    c = Ck.reshape(K, D)
    tm = _TM if M % _TM == 0 else M
    out = pl.pallas_call(
        _dist_kernel,
        grid=(M // tm,),
        in_specs=[
            pl.BlockSpec((tm, D), lambda i: (i, 0)),
            pl.BlockSpec((K, D), lambda i: (0, 0)),
        ],
        out_specs=pl.BlockSpec((tm, K), lambda i: (i, 0)),
        out_shape=jax.ShapeDtypeStruct((M, K), jnp.float32),
    )(f, c)
    return out.reshape(B, S, K)


# speedup vs baseline: 1.0176x; 1.0176x over previous
import jax
import jax.numpy as jnp
from jax.experimental import pallas as pl
from jax.experimental.pallas import tpu as pltpu


_TM = 2304


def _dist_kernel(f_ref, c_ref, o_ref, cbf_ref, csq_ref):
    @pl.when(pl.program_id(0) == 0)
    def _():
        c = c_ref[...]
        cbf_ref[...] = c.astype(jnp.bfloat16)
        csq_ref[...] = jnp.sum(c * c, axis=1, keepdims=True).reshape(1, -1)

    f = f_ref[...]
    f_sq = jnp.sum(f * f, axis=1, keepdims=True)
    fneg = (-2.0 * f).astype(jnp.bfloat16)
    cross = jax.lax.dot_general(
        fneg, cbf_ref[...],
        dimension_numbers=(((1,), (1,)), ((), ())),
        preferred_element_type=jnp.float32)
    o_ref[...] = cross + f_sq + csq_ref[...]


def kernel(x, Ck):
    B, S, D = x.shape
    K = Ck.shape[1]
    M = B * S
    f = x.reshape(M, D)
    c = Ck.reshape(K, D)
    tm = _TM if M % _TM == 0 else M
    out = pl.pallas_call(
        _dist_kernel,
        grid=(M // tm,),
        in_specs=[
            pl.BlockSpec((tm, D), lambda i: (i, 0)),
            pl.BlockSpec((K, D), lambda i: (0, 0)),
        ],
        out_specs=pl.BlockSpec((tm, K), lambda i: (i, 0)),
        out_shape=jax.ShapeDtypeStruct((M, K), jnp.float32),
        scratch_shapes=[
            pltpu.VMEM((K, D), jnp.bfloat16),
            pltpu.VMEM((1, K), jnp.float32),
        ],
    )(f, c)
    return out.reshape(B, S, K)
